# Initial kernel scaffold; baseline (speedup 1.0000x reference)
import jax
import jax.numpy as jnp
from jax.experimental import pallas as pl


def kernel(h, edge_index, r):
    raise NotImplementedError("write your pallas kernel here")


# speedup vs baseline: 2.9654x; 2.9654x over previous
import functools

import jax
import jax.numpy as jnp
from jax import lax
from jax.experimental import pallas as pl
from jax.experimental.pallas import tpu as pltpu
from jax.experimental.pallas import tpu_sc as plsc

_N = 10000
_D = 128
_E = 320000
_NC = 2
_NS = 16
_NW = _NC * _NS
_PER_W = _E // _NW
_C = 80
_CHUNKS = _PER_W // _C
_G = _C // 16
_K = _D // 16


def _hr_body(h_ref, r_ref, o_ref):
    o_ref[:, :] = h_ref[:, :] * r_ref[:, :]


def _weight_rows(h, r):
    return pl.pallas_call(
        _hr_body,
        out_shape=jax.ShapeDtypeStruct((_N, _D), jnp.float32),
    )(h, r.reshape(1, _D))


def _edge_dot_body(hr_hbm, h_hbm, src_hbm, dst_hbm, out_hbm,
                   sidx, didx, srows, drows, qbuf, obuf, sem):
    wid = lax.axis_index("s") * _NC + lax.axis_index("c")
    base = wid * _PER_W
    pltpu.sync_copy(src_hbm.at[pl.ds(base, _PER_W)], sidx)
    pltpu.sync_copy(dst_hbm.at[pl.ds(base, _PER_W)], didx)

    lane = lax.iota(jnp.int32, 16)

    def chunk_body(c, carry):
        off = pl.multiple_of(c * _C, _C)
        cp_s = pltpu.async_copy(hr_hbm.at[sidx.at[pl.ds(off, _C)]], srows, sem)
        cp_d = pltpu.async_copy(h_hbm.at[didx.at[pl.ds(off, _C)]], drows, sem)
        cp_s.wait()
        cp_d.wait()
        for g in range(_G):
            for j in range(16):
                e = g * 16 + j
                acc = srows[e, pl.ds(0, 16)] * drows[e, pl.ds(0, 16)]
                for k in range(1, _K):
                    acc = acc + (srows[e, pl.ds(k * 16, 16)]
                                 * drows[e, pl.ds(k * 16, 16)])
                qbuf[j, :] = acc
            s = plsc.load_gather(qbuf, [lane, jnp.zeros((16,), jnp.int32)])
            for l in range(1, 16):
                s = s + plsc.load_gather(
                    qbuf, [lane, jnp.full((16,), l, jnp.int32)])
            obuf[pl.ds(off + g * 16, 16)] = s
        return carry

    lax.fori_loop(0, _CHUNKS, chunk_body, 0)
    pltpu.sync_copy(obuf, out_hbm.at[pl.ds(base, _PER_W)])


@functools.partial(jax.jit, donate_argnums=())
def _edge_scores(hr, h, src, dst):
    mesh = plsc.VectorSubcoreMesh(core_axis_name="c", subcore_axis_name="s")
    k = pl.kernel(
        _edge_dot_body,
        out_type=jax.ShapeDtypeStruct((_E,), jnp.float32),
        mesh=mesh,
        compiler_params=pltpu.CompilerParams(needs_layout_passes=False),
        scratch_types=[
            pltpu.VMEM((_PER_W,), jnp.int32),
            pltpu.VMEM((_PER_W,), jnp.int32),
            pltpu.VMEM((_C, _D), jnp.float32),
            pltpu.VMEM((_C, _D), jnp.float32),
            pltpu.VMEM((16, 16), jnp.float32),
            pltpu.VMEM((_PER_W,), jnp.float32),
            pltpu.SemaphoreType.DMA,
        ],
    )
    return k(hr, h, src, dst)


def kernel(h, edge_index, r):
    hr = _weight_rows(h, r)
    src = edge_index[0]
    dst = edge_index[1]
    return _edge_scores(hr, h, src, dst)


# speedup vs baseline: 6.7135x; 2.2639x over previous
import functools

import jax
import jax.numpy as jnp
from jax import lax
from jax.experimental import pallas as pl
from jax.experimental.pallas import tpu as pltpu
from jax.experimental.pallas import tpu_sc as plsc

_N = 10000
_D = 128
_E = 320000
_NC = 2
_NS = 16
_NW = _NC * _NS
_PER_W = _E // _NW
_C = 80
_CHUNKS = _PER_W // _C
_G = _C // 16
_K = _D // 16


def _hr_body(h_ref, r_ref, o_ref):
    o_ref[:, :] = h_ref[:, :] * r_ref[:, :]


def _weight_rows(h, r):
    return pl.pallas_call(
        _hr_body,
        out_shape=jax.ShapeDtypeStruct((_N, _D), jnp.float32),
    )(h, r.reshape(1, _D))


def _edge_dot_body(hr_hbm, h_hbm, src_hbm, dst_hbm, out_hbm,
                   sidx, didx, srows0, drows0, srows1, drows1,
                   qbuf, obuf, sem0, sem1):
    wid = lax.axis_index("s") * _NC + lax.axis_index("c")
    base = wid * _PER_W
    pltpu.sync_copy(src_hbm.at[pl.ds(base, _PER_W)], sidx)
    pltpu.sync_copy(dst_hbm.at[pl.ds(base, _PER_W)], didx)

    lane = lax.iota(jnp.int32, 16)

    def fire(off, sbuf, dbuf, sem):
        pltpu.async_copy(hr_hbm.at[sidx.at[pl.ds(off, _C)]], sbuf, sem)
        pltpu.async_copy(h_hbm.at[didx.at[pl.ds(off, _C)]], dbuf, sem)

    def drain(sbuf, dbuf, sem):
        pltpu.make_async_copy(hr_hbm.at[sidx.at[pl.ds(0, _C)]],
                              sbuf, sem).wait()
        pltpu.make_async_copy(h_hbm.at[didx.at[pl.ds(0, _C)]],
                              dbuf, sem).wait()

    def compute(coff, sbuf, dbuf):
        def group_body(g, carry):
            e0 = g * 16
            for j in range(16):
                e = e0 + j
                acc = sbuf[e, pl.ds(0, 16)] * dbuf[e, pl.ds(0, 16)]
                for k in range(1, _K):
                    acc = acc + (sbuf[e, pl.ds(k * 16, 16)]
                                 * dbuf[e, pl.ds(k * 16, 16)])
                qbuf[j, :] = acc
            s = plsc.load_gather(qbuf, [lane, jnp.zeros((16,), jnp.int32)])
            for l in range(1, 16):
                s = s + plsc.load_gather(
                    qbuf, [lane, jnp.full((16,), l, jnp.int32)])
            obuf[pl.ds(pl.multiple_of(coff + e0, 16), 16)] = s
            return carry
        lax.fori_loop(0, _G, group_body, 0)

    fire(0, srows0, drows0, sem0)

    def pair_body(i, carry):
        off0 = pl.multiple_of(i * 2 * _C, _C)
        fire(off0 + _C, srows1, drows1, sem1)
        drain(srows0, drows0, sem0)
        compute(off0, srows0, drows0)
        fire(off0 + 2 * _C, srows0, drows0, sem0)
        drain(srows1, drows1, sem1)
        compute(off0 + _C, srows1, drows1)
        return carry

    lax.fori_loop(0, (_CHUNKS - 1) // 2, pair_body, 0)
    drain(srows0, drows0, sem0)
    compute((_CHUNKS - 1) * _C, srows0, drows0)

    pltpu.sync_copy(obuf, out_hbm.at[pl.ds(base, _PER_W)])


@functools.partial(jax.jit, donate_argnums=())
def _edge_scores(hr, h, src, dst):
    mesh = plsc.VectorSubcoreMesh(core_axis_name="c", subcore_axis_name="s")
    k = pl.kernel(
        _edge_dot_body,
        out_type=jax.ShapeDtypeStruct((_E,), jnp.float32),
        mesh=mesh,
        compiler_params=pltpu.CompilerParams(needs_layout_passes=False),
        scratch_types=[
            pltpu.VMEM((_PER_W,), jnp.int32),
            pltpu.VMEM((_PER_W,), jnp.int32),
            pltpu.VMEM((_C, _D), jnp.float32),
            pltpu.VMEM((_C, _D), jnp.float32),
            pltpu.VMEM((_C, _D), jnp.float32),
            pltpu.VMEM((_C, _D), jnp.float32),
            pltpu.VMEM((16, 16), jnp.float32),
            pltpu.VMEM((_PER_W,), jnp.float32),
            pltpu.SemaphoreType.DMA,
            pltpu.SemaphoreType.DMA,
        ],
    )
    return k(hr, h, src, dst)


def kernel(h, edge_index, r):
    hr = _weight_rows(h, r)
    src = edge_index[0]
    dst = edge_index[1]
    return _edge_scores(hr, h, src, dst)
